# baseline (speedup 1.0000x reference)
import functools
import numpy as np
import jax
import jax.numpy as jnp
from jax import lax
from jax.experimental import pallas as pl
from jax.experimental.pallas import tpu as pltpu
from jax.experimental.pallas import tpu_sc as plsc

_NS = 8
_NPAM = 4
_K = 64
_CONST = (np.array([-3.0, -1.0, 1.0, 3.0], dtype=np.float32)
          * np.float32(1.0 / np.sqrt(10.0)))
_BSC = 256

_SC_NC, _SC_NSUB = 2, 16
_NW = _SC_NC * _SC_NSUB
_ISQ10 = float(1.0 / np.sqrt(10.0))
_CONSTF = tuple(float(c) for c in
                np.array([-3.0, -1.0, 1.0, 3.0], np.float64) * _ISQ10)
_BIG = 1e9
_CLIP = 20.0


def _vreg_ce(dv, pv, r0, r1, asc):
    a, b = dv[r0], dv[r1]
    pa, pb = pv[r0], pv[r1]
    le = a <= b
    mn = jnp.where(le, a, b)
    mx = jnp.where(le, b, a)
    pmn = jnp.where(le, pa, pb)
    pmx = jnp.where(le, pb, pa)
    if asc:
        dv[r0], dv[r1], pv[r0], pv[r1] = mn, mx, pmn, pmx
    else:
        dv[r0], dv[r1], pv[r0], pv[r1] = mx, mn, pmx, pmn


def _lane_ce(x, v, d, km_i):
    iota = lax.iota(jnp.int32, 16)
    x2 = _lane_gather(x, iota ^ d)
    v2 = _lane_gather(v, iota ^ d)
    le = x <= x2
    ge = x >= x2
    mn = jnp.where(le, x, x2)
    pmn = jnp.where(le, v, v2)
    mx = jnp.where(ge, x, x2)
    pmx = jnp.where(ge, v, v2)
    km = km_i == 1
    return jnp.where(km, mn, mx), jnp.where(km, pmn, pmx)


def _km_mask(r, k, d, asc=None):
    iota = lax.iota(jnp.int32, 16)
    s = d.bit_length() - 1
    is_lo = 1 - ((iota >> s) & 1)
    if asc is None:
        a = 1 - (((iota + 16 * r) >> k) & 1)
        return 1 - (a ^ is_lo)
    return is_lo if asc else 1 - is_lo


def _sc_select64(dv, pv):
    dv = list(dv)
    pv = list(pv)
    for k in range(2, 7):
        d = 2 ** (k - 1)
        while d >= 1:
            if d >= 16:
                dvr = d // 16
                for r in range(16):
                    if r & dvr:
                        continue
                    asc = (((r * 16) >> k) & 1) == 0
                    _vreg_ce(dv, pv, r, r + dvr, asc)
            else:
                for r in range(16):
                    km = _km_mask(r, k, d)
                    dv[r], pv[r] = _lane_ce(dv[r], pv[r], d, km)
            d //= 2
    sd, sp = [], []
    for base in (0, 8):
        for r in range(base, base + 4):
            a, b = dv[r], dv[r + 4]
            pa, pb = pv[r], pv[r + 4]
            le = a <= b
            sd.append(jnp.where(le, a, b))
            sp.append(jnp.where(le, pa, pb))
    for base, asc in ((0, True), (4, False)):
        _vreg_ce(sd, sp, base + 0, base + 2, asc)
        _vreg_ce(sd, sp, base + 1, base + 3, asc)
        _vreg_ce(sd, sp, base + 0, base + 1, asc)
        _vreg_ce(sd, sp, base + 2, base + 3, asc)
        for d in (8, 4, 2, 1):
            km = _km_mask(0, 0, d, asc=asc)
            for r in range(base, base + 4):
                sd[r], sp[r] = _lane_ce(sd[r], sp[r], d, km)
    od, op = [], []
    for r in range(4):
        a, b = sd[r], sd[r + 4]
        pa, pb = sp[r], sp[r + 4]
        le = a <= b
        od.append(jnp.where(le, a, b))
        op.append(jnp.where(le, pa, pb))
    return od, op


_GATHER_DNUMS = lax.GatherDimensionNumbers(
    offset_dims=(), collapsed_slice_dims=(0,), start_index_map=(0,))


def _lane_gather(x, idx):
    return lax.gather(x, idx[:, None], _GATHER_DNUMS, slice_sizes=(1,),
                      mode=lax.GatherScatterMode.PROMISE_IN_BOUNDS)


def _vred(x, op):
    iota = lax.iota(jnp.int32, 16)
    for k in (8, 4, 2, 1):
        x = op(x, _lane_gather(x, iota ^ k))
    return x


def _vsum(x):
    return _vred(x, jnp.add)


def _vmin(x):
    return _vred(x, jnp.minimum)


def _example_body(ex, hv, yv, sv, ov):
    i32 = jnp.int32
    f32 = jnp.float32
    hb = ex * 128
    yb = ex * 16
    s_ex = sv[pl.ds(yb, 16)]
    y_ex = yv[pl.ds(yb, 16)]
    sinv = 1.0 / s_ex
    hcol = [hv[pl.ds(hb + i * 16, 16)] for i in range(8)]
    nrm = [_vsum(hcol[i] * hcol[i] * sinv) for i in range(8)]
    one = lax.iota(i32, 16) * 0 + 1
    zero = one * 0
    rank = []
    for i in range(8):
        r = None
        for j in range(8):
            if j == i:
                continue
            c = (nrm[j] >= nrm[i]) if j < i else (nrm[j] > nrm[i])
            ci = jnp.where(c, one, zero)
            r = ci if r is None else r + ci
        rank.append(r)
    hp = []
    for p in range(8):
        acc = None
        for i in range(8):
            sel = rank[i] == p
            acc = (jnp.where(sel, hcol[i], 0.0) if acc is None
                   else jnp.where(sel, hcol[i], acc))
        hp.append(acc)
    hpd = [hp[p] * sinv for p in range(8)]
    Gp = {}
    for i in range(8):
        for j in range(i + 1):
            Gp[(i, j)] = _vsum(hpd[i] * hp[j])
    zp = [_vsum(hpd[i] * y_ex) for i in range(8)]
    L = [[None] * 8 for _ in range(8)]
    D = [None] * 8
    for j in range(8):
        acc = Gp[(j, j)]
        for k in range(j):
            acc = acc - L[j][k] * L[j][k] * D[k]
        D[j] = acc
        for i in range(j + 1, 8):
            a2 = Gp[(i, j)]
            for k in range(j):
                a2 = a2 - L[i][k] * L[j][k] * D[k]
            L[i][j] = a2 / D[j]
    u = [None] * 8
    for i in range(8):
        acc = zp[i]
        for k in range(i):
            acc = acc - L[i][k] * u[k]
        u[i] = acc
    v = [u[i] / D[i] for i in range(8)]
    iota = lax.iota(i32, 16)
    c6 = iota & 3
    c7 = iota >> 2
    s7 = (2.0 * c7.astype(f32) - 3.0) * _ISQ10
    s6 = (2.0 * c6.astype(f32) - 3.0) * _ISQ10
    r7 = v[7] - s7
    r6 = v[6] - L[7][6] * s7 - s6
    dist16 = D[7] * r7 * r7 + D[6] * r6 * r6
    pack16 = (c7 << 14) | (c6 << 12)
    m5 = L[7][5] * s7 + L[6][5] * s6
    resid5 = v[5] - m5
    dist4, pack4 = [], []
    for c in range(4):
        t5 = resid5 - _CONSTF[c]
        dist4.append(dist16 + D[5] * t5 * t5)
        pack4.append(pack16 | i32(c << 10))
    cvec = iota & 3
    cf_gen = (2.0 * cvec.astype(f32) - 3.0) * _ISQ10
    for si in range(4, -1, -1):
        chd, chp = [], []
        cshift = cvec << (2 * si)
        for q in range(4):
            m = None
            for j in range(si + 1, 8):
                ind = (pack4[q] >> (2 * j)) & 3
                symf = (2.0 * ind.astype(f32) - 3.0) * _ISQ10
                t = L[j][si] * symf
                m = t if m is None else m + t
            resid = v[si] - m
            base = dist4[q]
            dd = D[si]
            for j in range(4):
                pidx = (iota >> 2) + (4 * j)
                rg = _lane_gather(resid, pidx)
                bg = _lane_gather(base, pidx)
                pg = _lane_gather(pack4[q], pidx)
                tc = rg - cf_gen
                chd.append(bg + dd * tc * tc)
                chp.append(pg | cshift)
        dist4, pack4 = _sc_select64(chd, chp)
    ind = [[(pack4[q] >> (2 * rank[j])) & 3 for q in range(4)]
           for j in range(8)]
    llr = []
    for i in range(4):
        qam = [ind[i][q] * 4 + ind[i + 4][q] for q in range(4)]
        for bit in range(4):
            d0 = None
            d1 = None
            for q in range(4):
                b = (qam[q] >> (3 - bit)) & 1
                m0 = jnp.where(b == 0, dist4[q], _BIG)
                m1 = jnp.where(b == 1, dist4[q], _BIG)
                d0 = m0 if d0 is None else jnp.minimum(d0, m0)
                d1 = m1 if d1 is None else jnp.minimum(d1, m1)
            llr.append(jnp.clip(_vmin(d0) - _vmin(d1), -_CLIP, _CLIP))
    outvec = jnp.zeros((16,), f32)
    for l in range(16):
        outvec = jnp.where(iota == l, llr[l], outvec)
    ov[pl.ds(yb, 16)] = outvec


def _sc_kernel(h_hbm, y_hbm, s_hbm, out_hbm, hv, yv, sv, ov, *, epw):
    wid = lax.axis_index("s") * _SC_NC + lax.axis_index("c")
    pltpu.sync_copy(h_hbm.at[pl.ds(wid * (epw * 128), epw * 128)], hv)
    pltpu.sync_copy(y_hbm.at[pl.ds(wid * (epw * 16), epw * 16)], yv)
    pltpu.sync_copy(s_hbm.at[pl.ds(wid * (epw * 16), epw * 16)], sv)

    def body(ex, carry):
        _example_body(ex, hv, yv, sv, ov)
        return carry

    lax.fori_loop(0, epw, body, 0)
    pltpu.sync_copy(ov, out_hbm.at[pl.ds(wid * (epw * 16), epw * 16)])


def _ce(d_arr, p_arr, dist, asc_mask=None):
    N, B = d_arr.shape
    nb = N // (2 * dist)
    d = d_arr.reshape(nb, 2, dist, B)
    p = p_arr.reshape(nb, 2, dist, B)
    a, b = d[:, 0], d[:, 1]
    pa, pb = p[:, 0], p[:, 1]
    le = a <= b
    if asc_mask is None:
        le_eff = le
    else:
        le_eff = le == asc_mask
    first = jnp.where(le_eff, a, b)
    second = jnp.where(le_eff, b, a)
    pfirst = jnp.where(le_eff, pa, pb)
    psecond = jnp.where(le_eff, pb, pa)
    d_out = jnp.stack([first, second], axis=1).reshape(N, B)
    p_out = jnp.stack([pfirst, psecond], axis=1).reshape(N, B)
    return d_out, p_out


def _blk_iota(nb, dist):
    return jax.lax.broadcasted_iota(jnp.int32, (nb, 1, 1), 0) * (2 * dist)


def _tc_select64(d_arr, p_arr):
    N = 256
    for k in range(2, 7):
        d = 2 ** (k - 1)
        while d >= 1:
            nb = N // (2 * d)
            asc = ((_blk_iota(nb, d) >> k) & 1) == 0
            d_arr, p_arr = _ce(d_arr, p_arr, d, asc)
            d //= 2
    d_arr, p_arr = _ce(d_arr, p_arr, 64)
    d2 = jnp.concatenate([d_arr[0:64], d_arr[128:192]], axis=0)
    p2 = jnp.concatenate([p_arr[0:64], p_arr[128:192]], axis=0)
    for d in (32, 16, 8, 4, 2, 1):
        nb = 128 // (2 * d)
        asc = _blk_iota(nb, d) < 64
        d2, p2 = _ce(d2, p2, d, asc)
    d2, p2 = _ce(d2, p2, 64)
    return d2[0:64], p2[0:64]


def _kbest_block(h_ref, y_ref, s_ref, out_ref):
    h = h_ref[...]
    y = y_ref[...]
    s = s_ref[...]
    Bb = y.shape[-1]
    sinv = 1.0 / s
    hd = h * sinv[:, None, :]
    G = jnp.zeros((8, 8, Bb), jnp.float32)
    z = jnp.zeros((8, Bb), jnp.float32)
    for t in range(16):
        G = G + hd[t][:, None, :] * h[t][None, :, :]
        z = z + hd[t] * y[t][None, :]
    n = jnp.stack([G[i, i] for i in range(8)], axis=0)
    jlt = (jax.lax.broadcasted_iota(jnp.int32, (8, 8, 1), 1)
           < jax.lax.broadcasted_iota(jnp.int32, (8, 8, 1), 0))
    gt = n[None, :, :] > n[:, None, :]
    eq = (n[None, :, :] == n[:, None, :]) & jlt
    rank = jnp.sum((gt | eq).astype(jnp.int32), axis=1)
    P = (rank[None, :, :] == jnp.arange(8, dtype=jnp.int32)[:, None, None]
         ).astype(jnp.float32)
    tmp = jnp.zeros((8, 8, Bb), jnp.float32)
    for i in range(8):
        tmp = tmp + P[:, i][:, None, :] * G[i][None, :, :]
    Gp = jnp.zeros((8, 8, Bb), jnp.float32)
    for j in range(8):
        Gp = Gp + tmp[:, j][:, None, :] * P[:, j][None, :, :]
    zp = jnp.zeros((8, Bb), jnp.float32)
    for i in range(8):
        zp = zp + P[:, i] * z[i][None, :]
    L = [[None] * 8 for _ in range(8)]
    D = [None] * 8
    for j in range(8):
        acc = Gp[j, j]
        for k in range(j):
            acc = acc - L[j][k] * L[j][k] * D[k]
        D[j] = acc
        for i in range(j + 1, 8):
            a2 = Gp[i, j]
            for k in range(j):
                a2 = a2 - L[i][k] * L[j][k] * D[k]
            L[i][j] = a2 / D[j]
    u = [None] * 8
    for i in range(8):
        acc = zp[i]
        for k in range(i):
            acc = acc - L[i][k] * u[k]
        u[i] = acc
    v = [u[i] / D[i] for i in range(8)]
    dists = jnp.zeros((1, Bb), jnp.float32)
    packed = jnp.zeros((1, Bb), jnp.int32)
    for stream in range(_NS):
        si = _NS - 1 - stream
        Pcur = dists.shape[0]
        m = jnp.zeros((Pcur, Bb), jnp.float32)
        for j in range(si + 1, 8):
            ind = (packed >> (2 * j)) & 3
            sym = (2.0 * ind.astype(jnp.float32) - 3.0) * _ISQ10
            m = m + L[j][si][None, :] * sym
        resid = v[si][None, :] - m
        dd = D[si][None, :]
        newd, newp = [], []
        for c in range(_NPAM):
            t = resid - _CONST[c]
            newd.append(dists + dd * t * t)
            newp.append(packed | np.int32(c << (2 * si)))
        if 4 * Pcur <= _K:
            dists = jnp.concatenate(newd, axis=0)
            packed = jnp.concatenate(newp, axis=0)
        else:
            d_e = jnp.stack(newd, axis=1).reshape(4 * Pcur, Bb)
            p_e = jnp.stack(newp, axis=1).reshape(4 * Pcur, Bb)
            dists, packed = _tc_select64(d_e, p_e)
    ind = []
    for j in range(8):
        ind.append((packed >> (2 * rank[j][None, :])) & 3)
    for i in range(4):
        qam = ind[i] * _NPAM + ind[i + 4]
        for bit in range(4):
            b = (qam >> (3 - bit)) & 1
            d0 = jnp.min(jnp.where(b == 0, dists, _BIG), axis=0)
            d1 = jnp.min(jnp.where(b == 1, dists, _BIG), axis=0)
            out_ref[i * 4 + bit, :] = jnp.clip(d0 - d1, -_CLIP, _CLIP)


_sc_mesh = plsc.VectorSubcoreMesh(core_axis_name="c", subcore_axis_name="s")


def _make_sc_call(b):
    epw = b // _NW
    return functools.partial(
        pl.kernel,
        mesh=_sc_mesh,
        out_type=jax.ShapeDtypeStruct((b * 16,), jnp.float32),
        scratch_types=[
            pltpu.VMEM((epw * 128,), jnp.float32),
            pltpu.VMEM((epw * 16,), jnp.float32),
            pltpu.VMEM((epw * 16,), jnp.float32),
            pltpu.VMEM((epw * 16,), jnp.float32),
        ],
    )(functools.partial(_sc_kernel, epw=epw))


_sc_call = _make_sc_call(_BSC)


def kernel(y, h, s_diag):
    B = y.shape[0]
    ysc, hsc, ssc = y[:_BSC], h[:_BSC], s_diag[:_BSC]
    out_sc = _sc_call(
        jnp.transpose(hsc, (0, 2, 1)).reshape(_BSC * 128),
        ysc.reshape(_BSC * 16),
        ssc.reshape(_BSC * 16),
    ).reshape(_BSC, 4, 4)
    Btc = B - _BSC
    ht = jnp.transpose(h[_BSC:], (1, 2, 0))
    yt = jnp.transpose(y[_BSC:], (1, 0))
    st = jnp.transpose(s_diag[_BSC:], (1, 0))
    out_tc = pl.pallas_call(
        _kbest_block,
        grid=(1,),
        in_specs=[
            pl.BlockSpec((16, 8, Btc), lambda i: (0, 0, i)),
            pl.BlockSpec((16, Btc), lambda i: (0, i)),
            pl.BlockSpec((16, Btc), lambda i: (0, i)),
        ],
        out_specs=pl.BlockSpec((16, Btc), lambda i: (0, i)),
        out_shape=jax.ShapeDtypeStruct((16, Btc), jnp.float32),
    )(ht, yt, st)
    out_tc = jnp.transpose(out_tc, (1, 0)).reshape(Btc, 4, 4)
    return jnp.concatenate([out_sc, out_tc], axis=0)


# speedup vs baseline: 1.0203x; 1.0203x over previous
import functools
import numpy as np
import jax
import jax.numpy as jnp
from jax import lax
from jax.experimental import pallas as pl
from jax.experimental.pallas import tpu as pltpu
from jax.experimental.pallas import tpu_sc as plsc

_NS = 8
_NPAM = 4
_K = 64
_CONST = (np.array([-3.0, -1.0, 1.0, 3.0], dtype=np.float32)
          * np.float32(1.0 / np.sqrt(10.0)))
_BSC = 512

_SC_NC, _SC_NSUB = 2, 16
_NW = _SC_NC * _SC_NSUB
_ISQ10 = float(1.0 / np.sqrt(10.0))
_CONSTF = tuple(float(c) for c in
                np.array([-3.0, -1.0, 1.0, 3.0], np.float64) * _ISQ10)
_BIG = 1e9
_CLIP = 20.0


def _vreg_ce(dv, pv, r0, r1, asc):
    a, b = dv[r0], dv[r1]
    pa, pb = pv[r0], pv[r1]
    le = a <= b
    mn = jnp.where(le, a, b)
    mx = jnp.where(le, b, a)
    pmn = jnp.where(le, pa, pb)
    pmx = jnp.where(le, pb, pa)
    if asc:
        dv[r0], dv[r1], pv[r0], pv[r1] = mn, mx, pmn, pmx
    else:
        dv[r0], dv[r1], pv[r0], pv[r1] = mx, mn, pmx, pmn


def _lane_ce(x, v, d, km_i):
    iota = lax.iota(jnp.int32, 16)
    x2 = _lane_gather(x, iota ^ d)
    v2 = _lane_gather(v, iota ^ d)
    le = x <= x2
    ge = x >= x2
    mn = jnp.where(le, x, x2)
    pmn = jnp.where(le, v, v2)
    mx = jnp.where(ge, x, x2)
    pmx = jnp.where(ge, v, v2)
    km = km_i == 1
    return jnp.where(km, mn, mx), jnp.where(km, pmn, pmx)


def _km_mask(r, k, d, asc=None):
    iota = lax.iota(jnp.int32, 16)
    s = d.bit_length() - 1
    is_lo = 1 - ((iota >> s) & 1)
    if asc is None:
        a = 1 - (((iota + 16 * r) >> k) & 1)
        return 1 - (a ^ is_lo)
    return is_lo if asc else 1 - is_lo


def _sc_select64(dv, pv):
    dv = list(dv)
    pv = list(pv)
    for k in range(2, 7):
        d = 2 ** (k - 1)
        while d >= 1:
            if d >= 16:
                dvr = d // 16
                for r in range(16):
                    if r & dvr:
                        continue
                    asc = (((r * 16) >> k) & 1) == 0
                    _vreg_ce(dv, pv, r, r + dvr, asc)
            else:
                for r in range(16):
                    km = _km_mask(r, k, d)
                    dv[r], pv[r] = _lane_ce(dv[r], pv[r], d, km)
            d //= 2
    sd, sp = [], []
    for base in (0, 8):
        for r in range(base, base + 4):
            a, b = dv[r], dv[r + 4]
            pa, pb = pv[r], pv[r + 4]
            le = a <= b
            sd.append(jnp.where(le, a, b))
            sp.append(jnp.where(le, pa, pb))
    for base, asc in ((0, True), (4, False)):
        _vreg_ce(sd, sp, base + 0, base + 2, asc)
        _vreg_ce(sd, sp, base + 1, base + 3, asc)
        _vreg_ce(sd, sp, base + 0, base + 1, asc)
        _vreg_ce(sd, sp, base + 2, base + 3, asc)
        for d in (8, 4, 2, 1):
            km = _km_mask(0, 0, d, asc=asc)
            for r in range(base, base + 4):
                sd[r], sp[r] = _lane_ce(sd[r], sp[r], d, km)
    od, op = [], []
    for r in range(4):
        a, b = sd[r], sd[r + 4]
        pa, pb = sp[r], sp[r + 4]
        le = a <= b
        od.append(jnp.where(le, a, b))
        op.append(jnp.where(le, pa, pb))
    return od, op


_GATHER_DNUMS = lax.GatherDimensionNumbers(
    offset_dims=(), collapsed_slice_dims=(0,), start_index_map=(0,))


def _lane_gather(x, idx):
    return lax.gather(x, idx[:, None], _GATHER_DNUMS, slice_sizes=(1,),
                      mode=lax.GatherScatterMode.PROMISE_IN_BOUNDS)


def _vred(x, op):
    iota = lax.iota(jnp.int32, 16)
    for k in (8, 4, 2, 1):
        x = op(x, _lane_gather(x, iota ^ k))
    return x


def _vsum(x):
    return _vred(x, jnp.add)


def _vmin(x):
    return _vred(x, jnp.minimum)


def _example_body(ex, hv, yv, sv, ov):
    i32 = jnp.int32
    f32 = jnp.float32
    hb = ex * 128
    yb = ex * 16
    s_ex = sv[pl.ds(yb, 16)]
    y_ex = yv[pl.ds(yb, 16)]
    sinv = 1.0 / s_ex
    hcol = [hv[pl.ds(hb + i * 16, 16)] for i in range(8)]
    nrm = [_vsum(hcol[i] * hcol[i] * sinv) for i in range(8)]
    one = lax.iota(i32, 16) * 0 + 1
    zero = one * 0
    rank = []
    for i in range(8):
        r = None
        for j in range(8):
            if j == i:
                continue
            c = (nrm[j] >= nrm[i]) if j < i else (nrm[j] > nrm[i])
            ci = jnp.where(c, one, zero)
            r = ci if r is None else r + ci
        rank.append(r)
    hp = []
    for p in range(8):
        acc = None
        for i in range(8):
            sel = rank[i] == p
            acc = (jnp.where(sel, hcol[i], 0.0) if acc is None
                   else jnp.where(sel, hcol[i], acc))
        hp.append(acc)
    hpd = [hp[p] * sinv for p in range(8)]
    Gp = {}
    for i in range(8):
        for j in range(i + 1):
            Gp[(i, j)] = _vsum(hpd[i] * hp[j])
    zp = [_vsum(hpd[i] * y_ex) for i in range(8)]
    L = [[None] * 8 for _ in range(8)]
    D = [None] * 8
    for j in range(8):
        acc = Gp[(j, j)]
        for k in range(j):
            acc = acc - L[j][k] * L[j][k] * D[k]
        D[j] = acc
        for i in range(j + 1, 8):
            a2 = Gp[(i, j)]
            for k in range(j):
                a2 = a2 - L[i][k] * L[j][k] * D[k]
            L[i][j] = a2 / D[j]
    u = [None] * 8
    for i in range(8):
        acc = zp[i]
        for k in range(i):
            acc = acc - L[i][k] * u[k]
        u[i] = acc
    v = [u[i] / D[i] for i in range(8)]
    iota = lax.iota(i32, 16)
    c6 = iota & 3
    c7 = iota >> 2
    s7 = (2.0 * c7.astype(f32) - 3.0) * _ISQ10
    s6 = (2.0 * c6.astype(f32) - 3.0) * _ISQ10
    r7 = v[7] - s7
    r6 = v[6] - L[7][6] * s7 - s6
    dist16 = D[7] * r7 * r7 + D[6] * r6 * r6
    pack16 = (c7 << 14) | (c6 << 12)
    m5 = L[7][5] * s7 + L[6][5] * s6
    resid5 = v[5] - m5
    dist4, pack4 = [], []
    for c in range(4):
        t5 = resid5 - _CONSTF[c]
        dist4.append(dist16 + D[5] * t5 * t5)
        pack4.append(pack16 | i32(c << 10))
    cvec = iota & 3
    cf_gen = (2.0 * cvec.astype(f32) - 3.0) * _ISQ10
    for si in range(4, -1, -1):
        chd, chp = [], []
        cshift = cvec << (2 * si)
        for q in range(4):
            m = None
            for j in range(si + 1, 8):
                ind = (pack4[q] >> (2 * j)) & 3
                symf = (2.0 * ind.astype(f32) - 3.0) * _ISQ10
                t = L[j][si] * symf
                m = t if m is None else m + t
            resid = v[si] - m
            base = dist4[q]
            dd = D[si]
            for j in range(4):
                pidx = (iota >> 2) + (4 * j)
                rg = _lane_gather(resid, pidx)
                bg = _lane_gather(base, pidx)
                pg = _lane_gather(pack4[q], pidx)
                tc = rg - cf_gen
                chd.append(bg + dd * tc * tc)
                chp.append(pg | cshift)
        dist4, pack4 = _sc_select64(chd, chp)
    ind = [[(pack4[q] >> (2 * rank[j])) & 3 for q in range(4)]
           for j in range(8)]
    llr = []
    for i in range(4):
        qam = [ind[i][q] * 4 + ind[i + 4][q] for q in range(4)]
        for bit in range(4):
            d0 = None
            d1 = None
            for q in range(4):
                b = (qam[q] >> (3 - bit)) & 1
                m0 = jnp.where(b == 0, dist4[q], _BIG)
                m1 = jnp.where(b == 1, dist4[q], _BIG)
                d0 = m0 if d0 is None else jnp.minimum(d0, m0)
                d1 = m1 if d1 is None else jnp.minimum(d1, m1)
            llr.append(jnp.clip(_vmin(d0) - _vmin(d1), -_CLIP, _CLIP))
    outvec = jnp.zeros((16,), f32)
    for l in range(16):
        outvec = jnp.where(iota == l, llr[l], outvec)
    ov[pl.ds(yb, 16)] = outvec


def _sc_kernel(h_hbm, y_hbm, s_hbm, out_hbm, hv, yv, sv, ov, *, epw):
    wid = lax.axis_index("s") * _SC_NC + lax.axis_index("c")
    pltpu.sync_copy(h_hbm.at[pl.ds(wid * (epw * 128), epw * 128)], hv)
    pltpu.sync_copy(y_hbm.at[pl.ds(wid * (epw * 16), epw * 16)], yv)
    pltpu.sync_copy(s_hbm.at[pl.ds(wid * (epw * 16), epw * 16)], sv)

    def body(ex, carry):
        _example_body(ex, hv, yv, sv, ov)
        return carry

    lax.fori_loop(0, epw, body, 0)
    pltpu.sync_copy(ov, out_hbm.at[pl.ds(wid * (epw * 16), epw * 16)])


def _ce(d_arr, p_arr, dist, asc_mask=None):
    N, B = d_arr.shape
    nb = N // (2 * dist)
    d = d_arr.reshape(nb, 2, dist, B)
    p = p_arr.reshape(nb, 2, dist, B)
    a, b = d[:, 0], d[:, 1]
    pa, pb = p[:, 0], p[:, 1]
    le = a <= b
    if asc_mask is None:
        le_eff = le
    else:
        le_eff = le == asc_mask
    first = jnp.where(le_eff, a, b)
    second = jnp.where(le_eff, b, a)
    pfirst = jnp.where(le_eff, pa, pb)
    psecond = jnp.where(le_eff, pb, pa)
    d_out = jnp.stack([first, second], axis=1).reshape(N, B)
    p_out = jnp.stack([pfirst, psecond], axis=1).reshape(N, B)
    return d_out, p_out


def _blk_iota(nb, dist):
    return jax.lax.broadcasted_iota(jnp.int32, (nb, 1, 1), 0) * (2 * dist)


def _tc_select64(d_arr, p_arr):
    N = 256
    for k in range(2, 7):
        d = 2 ** (k - 1)
        while d >= 1:
            nb = N // (2 * d)
            asc = ((_blk_iota(nb, d) >> k) & 1) == 0
            d_arr, p_arr = _ce(d_arr, p_arr, d, asc)
            d //= 2
    d_arr, p_arr = _ce(d_arr, p_arr, 64)
    d2 = jnp.concatenate([d_arr[0:64], d_arr[128:192]], axis=0)
    p2 = jnp.concatenate([p_arr[0:64], p_arr[128:192]], axis=0)
    for d in (32, 16, 8, 4, 2, 1):
        nb = 128 // (2 * d)
        asc = _blk_iota(nb, d) < 64
        d2, p2 = _ce(d2, p2, d, asc)
    d2, p2 = _ce(d2, p2, 64)
    return d2[0:64], p2[0:64]


def _kbest_block(h_ref, y_ref, s_ref, out_ref):
    h = h_ref[...]
    y = y_ref[...]
    s = s_ref[...]
    Bb = y.shape[-1]
    sinv = 1.0 / s
    hd = h * sinv[:, None, :]
    G = jnp.zeros((8, 8, Bb), jnp.float32)
    z = jnp.zeros((8, Bb), jnp.float32)
    for t in range(16):
        G = G + hd[t][:, None, :] * h[t][None, :, :]
        z = z + hd[t] * y[t][None, :]
    n = jnp.stack([G[i, i] for i in range(8)], axis=0)
    jlt = (jax.lax.broadcasted_iota(jnp.int32, (8, 8, 1), 1)
           < jax.lax.broadcasted_iota(jnp.int32, (8, 8, 1), 0))
    gt = n[None, :, :] > n[:, None, :]
    eq = (n[None, :, :] == n[:, None, :]) & jlt
    rank = jnp.sum((gt | eq).astype(jnp.int32), axis=1)
    P = (rank[None, :, :] == jnp.arange(8, dtype=jnp.int32)[:, None, None]
         ).astype(jnp.float32)
    tmp = jnp.zeros((8, 8, Bb), jnp.float32)
    for i in range(8):
        tmp = tmp + P[:, i][:, None, :] * G[i][None, :, :]
    Gp = jnp.zeros((8, 8, Bb), jnp.float32)
    for j in range(8):
        Gp = Gp + tmp[:, j][:, None, :] * P[:, j][None, :, :]
    zp = jnp.zeros((8, Bb), jnp.float32)
    for i in range(8):
        zp = zp + P[:, i] * z[i][None, :]
    L = [[None] * 8 for _ in range(8)]
    D = [None] * 8
    for j in range(8):
        acc = Gp[j, j]
        for k in range(j):
            acc = acc - L[j][k] * L[j][k] * D[k]
        D[j] = acc
        for i in range(j + 1, 8):
            a2 = Gp[i, j]
            for k in range(j):
                a2 = a2 - L[i][k] * L[j][k] * D[k]
            L[i][j] = a2 / D[j]
    u = [None] * 8
    for i in range(8):
        acc = zp[i]
        for k in range(i):
            acc = acc - L[i][k] * u[k]
        u[i] = acc
    v = [u[i] / D[i] for i in range(8)]
    dists = jnp.zeros((1, Bb), jnp.float32)
    packed = jnp.zeros((1, Bb), jnp.int32)
    for stream in range(_NS):
        si = _NS - 1 - stream
        Pcur = dists.shape[0]
        m = jnp.zeros((Pcur, Bb), jnp.float32)
        for j in range(si + 1, 8):
            ind = (packed >> (2 * j)) & 3
            sym = (2.0 * ind.astype(jnp.float32) - 3.0) * _ISQ10
            m = m + L[j][si][None, :] * sym
        resid = v[si][None, :] - m
        dd = D[si][None, :]
        newd, newp = [], []
        for c in range(_NPAM):
            t = resid - _CONST[c]
            newd.append(dists + dd * t * t)
            newp.append(packed | np.int32(c << (2 * si)))
        if 4 * Pcur <= _K:
            dists = jnp.concatenate(newd, axis=0)
            packed = jnp.concatenate(newp, axis=0)
        else:
            d_e = jnp.stack(newd, axis=1).reshape(4 * Pcur, Bb)
            p_e = jnp.stack(newp, axis=1).reshape(4 * Pcur, Bb)
            dists, packed = _tc_select64(d_e, p_e)
    ind = []
    for j in range(8):
        ind.append((packed >> (2 * rank[j][None, :])) & 3)
    for i in range(4):
        qam = ind[i] * _NPAM + ind[i + 4]
        for bit in range(4):
            b = (qam >> (3 - bit)) & 1
            d0 = jnp.min(jnp.where(b == 0, dists, _BIG), axis=0)
            d1 = jnp.min(jnp.where(b == 1, dists, _BIG), axis=0)
            out_ref[i * 4 + bit, :] = jnp.clip(d0 - d1, -_CLIP, _CLIP)


_sc_mesh = plsc.VectorSubcoreMesh(core_axis_name="c", subcore_axis_name="s")


def _make_sc_call(b):
    epw = b // _NW
    return functools.partial(
        pl.kernel,
        mesh=_sc_mesh,
        out_type=jax.ShapeDtypeStruct((b * 16,), jnp.float32),
        scratch_types=[
            pltpu.VMEM((epw * 128,), jnp.float32),
            pltpu.VMEM((epw * 16,), jnp.float32),
            pltpu.VMEM((epw * 16,), jnp.float32),
            pltpu.VMEM((epw * 16,), jnp.float32),
        ],
    )(functools.partial(_sc_kernel, epw=epw))


_sc_call = _make_sc_call(_BSC)


def kernel(y, h, s_diag):
    B = y.shape[0]
    ysc, hsc, ssc = y[:_BSC], h[:_BSC], s_diag[:_BSC]
    out_sc = _sc_call(
        jnp.transpose(hsc, (0, 2, 1)).reshape(_BSC * 128),
        ysc.reshape(_BSC * 16),
        ssc.reshape(_BSC * 16),
    ).reshape(_BSC, 4, 4)
    Btc = B - _BSC
    ht = jnp.transpose(h[_BSC:], (1, 2, 0))
    yt = jnp.transpose(y[_BSC:], (1, 0))
    st = jnp.transpose(s_diag[_BSC:], (1, 0))
    out_tc = pl.pallas_call(
        _kbest_block,
        grid=(1,),
        in_specs=[
            pl.BlockSpec((16, 8, Btc), lambda i: (0, 0, i)),
            pl.BlockSpec((16, Btc), lambda i: (0, i)),
            pl.BlockSpec((16, Btc), lambda i: (0, i)),
        ],
        out_specs=pl.BlockSpec((16, Btc), lambda i: (0, i)),
        out_shape=jax.ShapeDtypeStruct((16, Btc), jnp.float32),
    )(ht, yt, st)
    out_tc = jnp.transpose(out_tc, (1, 0)).reshape(Btc, 4, 4)
    return jnp.concatenate([out_sc, out_tc], axis=0)
